# baseline (speedup 1.0000x reference)
import functools

import jax
import jax.numpy as jnp
from jax import lax
from jax.experimental import pallas as pl
from jax.experimental.pallas import tpu as pltpu
from jax.experimental.pallas import tpu_sc as plsc

H_BEV = 200
W_BEV = 200
RES = 0.5
HF, WF = 32, 88
HP, WP = HF + 2, WF + 2
NV = HF * WF
CHUNK = 8 * W_BEV
NCHUNK = (H_BEV * W_BEV) // CHUNK


def _lift_kernel(kinv_ref, up_ref, vp_ref, pts_ref):
    up = up_ref[...]
    vp = vp_ref[...]
    k = kinv_ref
    pts_ref[0] = up * k[0, 0] + vp * k[0, 1] + k[0, 2]
    pts_ref[1] = up * k[0, 3] + vp * k[0, 4] + k[0, 5]
    pts_ref[2] = up * k[0, 6] + vp * k[0, 7] + k[0, 8]


def _head_kernel(xp_ref, w9_ref, b1_ref, gamma_ref, beta_ref, w2_ref, b2_ref,
                 bins_ref, kinv_ref, uvh_ref,
                 dist_ref, expd_ref, dsc_ref, flats_ref):
    acc = jnp.zeros((NV, 128), jnp.float32)
    for dy in range(3):
        for dx in range(3):
            xs = xp_ref[dy:dy + HF, dx:dx + WF, :].reshape(NV, 256)
            acc = acc + jnp.dot(xs, w9_ref[dy * 3 + dx],
                                preferred_element_type=jnp.float32)
    h = acc + b1_ref[0, :][None, :]
    inv_n = 1.0 / NV
    mu = jnp.sum(h, axis=0, keepdims=True) * inv_n
    var = jnp.sum((h - mu) ** 2, axis=0, keepdims=True) * inv_n
    hn = (h - mu) / jnp.sqrt(var + 1e-5)
    hn = hn * gamma_ref[0, :][None, :] + beta_ref[0, :][None, :]
    hn = jnp.maximum(hn, 0.0)
    logits = jnp.dot(hn, w2_ref[...], preferred_element_type=jnp.float32)
    logits = logits + b2_ref[0, :][None, :]
    mx = jnp.max(logits, axis=1, keepdims=True)
    e = jnp.exp(logits - mx)
    dist = e / jnp.sum(e, axis=1, keepdims=True)

    dist3 = dist.reshape(HF, WF, 64)
    expd_ref[...] = jnp.sum(dist3 * bins_ref[0, :][None, None, :], axis=2)
    for y in range(HF):
        dist_ref[:, y, :] = dist3[y].T
    d = jnp.sum(dist) * (inv_n / 64.0)
    dsc_ref[...] = d.reshape(1, 1)

    x_min = -(H_BEV * RES / 2.0)
    k = kinv_ref
    u0, v0 = uvh_ref[0, 0], uvh_ref[0, 1]
    u1, v1 = uvh_ref[0, 2], uvh_ref[0, 3]
    flat = [None] * 3
    for j in range(3):
        px = (u0 * k[0, 3 * j] + v0 * k[0, 3 * j + 1] + k[0, 3 * j + 2]) * d
        py = (u1 * k[0, 3 * j] + v1 * k[0, 3 * j + 1] + k[0, 3 * j + 2]) * d
        gx = jnp.clip(((px - x_min) / RES).astype(jnp.int32), 0, W_BEV - 1)
        gy = jnp.clip(((py - x_min) / RES).astype(jnp.int32), 0, H_BEV - 1)
        flat[j] = gy * W_BEV + gx
    lane = jax.lax.broadcasted_iota(jnp.int32, (1, 48), 1)
    flats_ref[...] = jnp.where(
        lane < 16, flat[0], jnp.where(lane < 32, flat[1], flat[2]))


def _sc_splat_body(flats_hbm, out_hbm, flats_v, buf_v):
    wid = lax.axis_index("s") * 2 + lax.axis_index("c")

    @pl.when(wid < NCHUNK)
    def _():
        pltpu.sync_copy(flats_hbm, flats_v)
        lane = lax.broadcasted_iota(jnp.int32, (16,), 0)
        b0 = flats_v[pl.ds(0, 16)]
        b1 = flats_v[pl.ds(16, 16)]
        b2 = flats_v[pl.ds(32, 16)]
        base = wid * CHUNK

        def _fill(i, carry):
            cell = base + i * 16 + lane
            m = (cell == b0) | (cell == b1) | (cell == b2)
            buf_v[pl.ds(i * 16, 16)] = jnp.where(m, 1.0, 0.0)
            return carry

        lax.fori_loop(0, CHUNK // 16, _fill, 0)
        pltpu.sync_copy(buf_v, out_hbm.at[pl.ds(base, CHUNK)])


def _sc_splat(flats):
    mesh = plsc.VectorSubcoreMesh(core_axis_name="c", subcore_axis_name="s")
    k = functools.partial(
        pl.kernel,
        mesh=mesh,
        out_type=jax.ShapeDtypeStruct((H_BEV * W_BEV,), jnp.float32),
        scratch_types=[
            pltpu.VMEM((48,), jnp.int32),
            pltpu.VMEM((CHUNK,), jnp.float32),
        ],
    )(_sc_splat_body)
    return k(flats)


def kernel(camera_features, pixels_uv, K_inv, conv1_w, conv1_b, bn_gamma,
           bn_beta, conv2_w, conv2_b, depth_bins):
    xp3 = jnp.zeros((HP, WP, 256), jnp.float32)
    xp3 = xp3.at[1:1 + HF, 1:1 + WF, :].set(camera_features[0].transpose(1, 2, 0))
    w9 = conv1_w.transpose(2, 3, 1, 0).reshape(9, 256, 128)
    w2 = conv2_w[:, :, 0, 0].T

    npts = pixels_uv.shape[1]
    up = pixels_uv[0, :, 0]
    vp = pixels_uv[0, :, 1]
    uvh = jnp.stack([up[0], vp[0], up[1], vp[1]]).reshape(1, 4)
    kinv = K_inv[0].reshape(1, 9)

    pts_nod = pl.pallas_call(
        _lift_kernel,
        out_shape=jax.ShapeDtypeStruct((3, npts), jnp.float32),
    )(kinv, up, vp)

    dist, expd, dsc, flats = pl.pallas_call(
        _head_kernel,
        out_shape=(
            jax.ShapeDtypeStruct((64, HF, WF), jnp.float32),
            jax.ShapeDtypeStruct((HF, WF), jnp.float32),
            jax.ShapeDtypeStruct((1, 1), jnp.float32),
            jax.ShapeDtypeStruct((1, 48), jnp.int32),
        ),
    )(xp3, w9, conv1_b.reshape(1, 128), bn_gamma.reshape(1, 128),
      bn_beta.reshape(1, 128), w2, conv2_b.reshape(1, 64),
      depth_bins.reshape(1, 64), kinv, uvh)

    depth_dist = dist[None]
    exp_depth = expd[None]
    pts3d = (pts_nod * dsc[0, 0]).T.reshape(1, 1, npts, 3)
    bev_grid = _sc_splat(flats.reshape(48)).reshape(1, H_BEV, W_BEV)
    return bev_grid, depth_dist, exp_depth, pts3d


# speedup vs baseline: 1.3484x; 1.3484x over previous
import functools

import jax
import jax.numpy as jnp
from jax import lax
from jax.experimental import pallas as pl
from jax.experimental.pallas import tpu as pltpu
from jax.experimental.pallas import tpu_sc as plsc

H_BEV = 200
W_BEV = 200
RES = 0.5
HF, WF = 32, 88
HP, WP = HF + 2, WF + 2
NV = HF * WF
CHUNK = 8 * W_BEV
NCHUNK = (H_BEV * W_BEV) // CHUNK


def _main_kernel(xp_ref, w9_ref, b1_ref, gamma_ref, beta_ref, w2_ref, b2_ref,
                 bins_ref, kinv_ref, uvh_ref, up_ref, vp_ref,
                 dist_ref, expd_ref, xo_ref, yo_ref, zo_ref, flats_ref):
    acc = jnp.zeros((NV, 128), jnp.float32)
    for dy in range(3):
        for dx in range(3):
            xs = xp_ref[dy:dy + HF, dx:dx + WF, :].reshape(NV, 256)
            acc = acc + jnp.dot(xs, w9_ref[dy * 3 + dx],
                                preferred_element_type=jnp.float32)
    h = acc + b1_ref[0, :][None, :]
    inv_n = 1.0 / NV
    mu = jnp.sum(h, axis=0, keepdims=True) * inv_n
    var = jnp.sum((h - mu) ** 2, axis=0, keepdims=True) * inv_n
    hn = (h - mu) / jnp.sqrt(var + 1e-5)
    hn = hn * gamma_ref[0, :][None, :] + beta_ref[0, :][None, :]
    hn = jnp.maximum(hn, 0.0)
    logits = jnp.dot(hn, w2_ref[...], preferred_element_type=jnp.float32)
    logits = logits + b2_ref[0, :][None, :]
    mx = jnp.max(logits, axis=1, keepdims=True)
    e = jnp.exp(logits - mx)
    dist = e / jnp.sum(e, axis=1, keepdims=True)

    dist3 = dist.reshape(HF, WF, 64)
    expd_ref[...] = jnp.sum(dist3 * bins_ref[0, :][None, None, :], axis=2)
    for y in range(HF):
        dist_ref[:, y, :] = dist3[y].T
    d = jnp.sum(dist) * (inv_n / 64.0)

    up = up_ref[...]
    vp = vp_ref[...]
    k = kinv_ref
    xo_ref[...] = (up * k[0, 0] + vp * k[0, 1] + k[0, 2]) * d
    yo_ref[...] = (up * k[0, 3] + vp * k[0, 4] + k[0, 5]) * d
    zo_ref[...] = (up * k[0, 6] + vp * k[0, 7] + k[0, 8]) * d

    x_min = -(H_BEV * RES / 2.0)
    u0, v0 = uvh_ref[0, 0], uvh_ref[0, 1]
    u1, v1 = uvh_ref[0, 2], uvh_ref[0, 3]
    flat = [None] * 3
    for j in range(3):
        px = (u0 * k[0, 3 * j] + v0 * k[0, 3 * j + 1] + k[0, 3 * j + 2]) * d
        py = (u1 * k[0, 3 * j] + v1 * k[0, 3 * j + 1] + k[0, 3 * j + 2]) * d
        gx = jnp.clip(((px - x_min) / RES).astype(jnp.int32), 0, W_BEV - 1)
        gy = jnp.clip(((py - x_min) / RES).astype(jnp.int32), 0, H_BEV - 1)
        flat[j] = gy * W_BEV + gx
    lane = jax.lax.broadcasted_iota(jnp.int32, (1, 48), 1)
    flats_ref[...] = jnp.where(
        lane < 16, flat[0], jnp.where(lane < 32, flat[1], flat[2]))


def _sc_splat_body(flats_hbm, out_hbm, flats_v, buf_v):
    wid = lax.axis_index("s") * 2 + lax.axis_index("c")

    @pl.when(wid < NCHUNK)
    def _():
        pltpu.sync_copy(flats_hbm, flats_v)
        lane = lax.broadcasted_iota(jnp.int32, (16,), 0)
        b0 = flats_v[pl.ds(0, 16)]
        b1 = flats_v[pl.ds(16, 16)]
        b2 = flats_v[pl.ds(32, 16)]
        base = wid * CHUNK

        def _fill(i, carry):
            cell = base + i * 16 + lane
            m = (cell == b0) | (cell == b1) | (cell == b2)
            buf_v[pl.ds(i * 16, 16)] = jnp.where(m, 1.0, 0.0)
            return carry

        lax.fori_loop(0, CHUNK // 16, _fill, 0)
        pltpu.sync_copy(buf_v, out_hbm.at[pl.ds(base, CHUNK)])


def _sc_splat(flats):
    mesh = plsc.VectorSubcoreMesh(core_axis_name="c", subcore_axis_name="s")
    k = functools.partial(
        pl.kernel,
        mesh=mesh,
        out_type=jax.ShapeDtypeStruct((H_BEV * W_BEV,), jnp.float32),
        scratch_types=[
            pltpu.VMEM((48,), jnp.int32),
            pltpu.VMEM((CHUNK,), jnp.float32),
        ],
    )(_sc_splat_body)
    return k(flats)


def kernel(camera_features, pixels_uv, K_inv, conv1_w, conv1_b, bn_gamma,
           bn_beta, conv2_w, conv2_b, depth_bins):
    xp3 = jnp.zeros((HP, WP, 256), jnp.float32)
    xp3 = xp3.at[1:1 + HF, 1:1 + WF, :].set(camera_features[0].transpose(1, 2, 0))
    w9 = conv1_w.transpose(2, 3, 1, 0).reshape(9, 256, 128)
    w2 = conv2_w[:, :, 0, 0].T

    npts = pixels_uv.shape[1]
    up = lax.slice(pixels_uv, (0, 0, 0), (1, npts, 1)).reshape(npts)
    vp = lax.slice(pixels_uv, (0, 0, 1), (1, npts, 2)).reshape(npts)
    uvh = jnp.stack([up[0], vp[0], up[1], vp[1]]).reshape(1, 4)
    kinv = K_inv[0].reshape(1, 9)

    dist, expd, xo, yo, zo, flats = pl.pallas_call(
        _main_kernel,
        out_shape=(
            jax.ShapeDtypeStruct((64, HF, WF), jnp.float32),
            jax.ShapeDtypeStruct((HF, WF), jnp.float32),
            jax.ShapeDtypeStruct((npts,), jnp.float32),
            jax.ShapeDtypeStruct((npts,), jnp.float32),
            jax.ShapeDtypeStruct((npts,), jnp.float32),
            jax.ShapeDtypeStruct((1, 48), jnp.int32),
        ),
    )(xp3, w9, conv1_b.reshape(1, 128), bn_gamma.reshape(1, 128),
      bn_beta.reshape(1, 128), w2, conv2_b.reshape(1, 64),
      depth_bins.reshape(1, 64), kinv, uvh, up, vp)

    depth_dist = dist[None]
    exp_depth = expd[None]
    pts3d = (jnp.concatenate([xo, yo, zo]).reshape(3, npts)
             .T.reshape(1, 1, npts, 3))
    bev_grid = _sc_splat(flats.reshape(48)).reshape(1, H_BEV, W_BEV)
    return bev_grid, depth_dist, exp_depth, pts3d


# speedup vs baseline: 1.7015x; 1.2619x over previous
import functools

import jax
import jax.numpy as jnp
from jax import lax
from jax.experimental import pallas as pl
from jax.experimental.pallas import tpu as pltpu
from jax.experimental.pallas import tpu_sc as plsc

H_BEV = 200
W_BEV = 200
RES = 0.5
HF, WF = 32, 88
HP, WP = HF + 2, WF + 2
NV = HF * WF
CHUNK = 8 * W_BEV
NCHUNK = (H_BEV * W_BEV) // CHUNK


def _main_kernel(xp_ref, w9_ref, b1_ref, gamma_ref, beta_ref, w2_ref, b2_ref,
                 bins_ref, kinv_ref, uvh_ref, up_ref, vp_ref,
                 dist_ref, expd_ref, xo_ref, yo_ref, zo_ref, flats_ref):
    acc = jnp.zeros((NV, 128), jnp.float32)
    for dy in range(3):
        for dx in range(3):
            xs = xp_ref[dy:dy + HF, dx:dx + WF, :].reshape(NV, 256)
            acc = acc + jnp.dot(xs, w9_ref[dy * 3 + dx],
                                preferred_element_type=jnp.float32)
    h = acc + b1_ref[0, :][None, :]
    inv_n = 1.0 / NV
    mu = jnp.sum(h, axis=0, keepdims=True) * inv_n
    var = jnp.sum((h - mu) ** 2, axis=0, keepdims=True) * inv_n
    hn = (h - mu) / jnp.sqrt(var + 1e-5)
    hn = hn * gamma_ref[0, :][None, :] + beta_ref[0, :][None, :]
    hn = jnp.maximum(hn, 0.0)
    logits = jnp.dot(hn, w2_ref[...], preferred_element_type=jnp.float32)
    logits = logits + b2_ref[0, :][None, :]
    mx = jnp.max(logits, axis=1, keepdims=True)
    e = jnp.exp(logits - mx)
    dist = e / jnp.sum(e, axis=1, keepdims=True)

    dist3 = dist.reshape(HF, WF, 64)
    expd_ref[...] = jnp.sum(dist3 * bins_ref[0, :][None, None, :], axis=2)
    for y in range(HF):
        dist_ref[:, y, :] = dist3[y].T
    d = jnp.sum(dist) * (inv_n / 64.0)

    up = up_ref[...]
    vp = vp_ref[...]
    k = kinv_ref
    xo_ref[...] = (up * k[0, 0] + vp * k[0, 1] + k[0, 2]) * d
    yo_ref[...] = (up * k[0, 3] + vp * k[0, 4] + k[0, 5]) * d
    zo_ref[...] = (up * k[0, 6] + vp * k[0, 7] + k[0, 8]) * d

    x_min = -(H_BEV * RES / 2.0)
    u0, v0 = uvh_ref[0, 0], uvh_ref[0, 1]
    u1, v1 = uvh_ref[0, 2], uvh_ref[0, 3]
    flat = [None] * 3
    for j in range(3):
        px = (u0 * k[0, 3 * j] + v0 * k[0, 3 * j + 1] + k[0, 3 * j + 2]) * d
        py = (u1 * k[0, 3 * j] + v1 * k[0, 3 * j + 1] + k[0, 3 * j + 2]) * d
        gx = jnp.clip(((px - x_min) / RES).astype(jnp.int32), 0, W_BEV - 1)
        gy = jnp.clip(((py - x_min) / RES).astype(jnp.int32), 0, H_BEV - 1)
        flat[j] = gy * W_BEV + gx
    lane = jax.lax.broadcasted_iota(jnp.int32, (1, 48), 1)
    flats_ref[...] = jnp.where(
        lane < 16, flat[0], jnp.where(lane < 32, flat[1], flat[2]))


def _sc_splat_body(flats_hbm, out_hbm, flats_v, buf_v):
    wid = lax.axis_index("s") * 2 + lax.axis_index("c")

    @pl.when(wid < NCHUNK)
    def _():
        pltpu.sync_copy(flats_hbm, flats_v)
        lane = lax.broadcasted_iota(jnp.int32, (16,), 0)
        b0 = flats_v[pl.ds(0, 16)]
        b1 = flats_v[pl.ds(16, 16)]
        b2 = flats_v[pl.ds(32, 16)]
        base = wid * CHUNK

        def _fill(i, carry):
            cell = base + i * 16 + lane
            m = (cell == b0) | (cell == b1) | (cell == b2)
            buf_v[pl.ds(i * 16, 16)] = jnp.where(m, 1.0, 0.0)
            return carry

        lax.fori_loop(0, CHUNK // 16, _fill, 0)
        pltpu.sync_copy(buf_v, out_hbm.at[pl.ds(base, CHUNK)])


def _sc_splat(flats):
    mesh = plsc.VectorSubcoreMesh(core_axis_name="c", subcore_axis_name="s")
    k = functools.partial(
        pl.kernel,
        mesh=mesh,
        out_type=jax.ShapeDtypeStruct((H_BEV * W_BEV,), jnp.float32),
        scratch_types=[
            pltpu.VMEM((48,), jnp.int32),
            pltpu.VMEM((CHUNK,), jnp.float32),
        ],
    )(_sc_splat_body)
    return k(flats)


def kernel(camera_features, pixels_uv, K_inv, conv1_w, conv1_b, bn_gamma,
           bn_beta, conv2_w, conv2_b, depth_bins):
    xp3 = jnp.zeros((HP, WP, 256), jnp.float32)
    xp3 = xp3.at[1:1 + HF, 1:1 + WF, :].set(camera_features[0].transpose(1, 2, 0))
    w9 = conv1_w.transpose(2, 3, 1, 0).reshape(9, 256, 128)
    w2 = conv2_w[:, :, 0, 0].T

    npts = pixels_uv.shape[1]
    up = pixels_uv[0, :, 0]
    vp = pixels_uv[0, :, 1]
    uvh = jnp.stack([up[0], vp[0], up[1], vp[1]]).reshape(1, 4)
    kinv = K_inv[0].reshape(1, 9)

    dist, expd, xo, yo, zo, flats = pl.pallas_call(
        _main_kernel,
        out_shape=(
            jax.ShapeDtypeStruct((64, HF, WF), jnp.float32),
            jax.ShapeDtypeStruct((HF, WF), jnp.float32),
            jax.ShapeDtypeStruct((npts,), jnp.float32),
            jax.ShapeDtypeStruct((npts,), jnp.float32),
            jax.ShapeDtypeStruct((npts,), jnp.float32),
            jax.ShapeDtypeStruct((1, 48), jnp.int32),
        ),
    )(xp3, w9, conv1_b.reshape(1, 128), bn_gamma.reshape(1, 128),
      bn_beta.reshape(1, 128), w2, conv2_b.reshape(1, 64),
      depth_bins.reshape(1, 64), kinv, uvh, up, vp)

    depth_dist = dist[None]
    exp_depth = expd[None]
    pts3d = jnp.stack([xo, yo, zo], axis=-1).reshape(1, 1, npts, 3)
    bev_grid = _sc_splat(flats.reshape(48)).reshape(1, H_BEV, W_BEV)
    return bev_grid, depth_dist, exp_depth, pts3d
